# Initial kernel scaffold; baseline (speedup 1.0000x reference)
import jax
import jax.numpy as jnp
from jax.experimental import pallas as pl


def kernel(words, head_pos, tail_pos, word_table, head_pos_table, tail_pos_table):
    raise NotImplementedError("write your pallas kernel here")


# speedup vs baseline: 3.8561x; 3.8561x over previous
import functools

import jax
import jax.numpy as jnp
from jax import lax
from jax.experimental import pallas as pl
from jax.experimental.pallas import tpu as pltpu
from jax.experimental.pallas import tpu_sc as plsc

NC, NS, L = 2, 16, 16
NW = NC * NS
B, SEQ = 4096, 200
N = B * SEQ
WD, PD, OD = 64, 16, 96
N_PER_W = N // NW
CHUNK = 1024
GSUB = CHUNK // 128
NCHUNK = N_PER_W // CHUNK
NROW128 = N // 128


@functools.partial(
    pl.kernel,
    out_type=jax.ShapeDtypeStruct((N, OD), jnp.float32),
    mesh=plsc.VectorSubcoreMesh(core_axis_name="c", subcore_axis_name="s"),
    scratch_types=[
        pltpu.VMEM((GSUB, 128), jnp.int32),
        pltpu.VMEM((GSUB, 128), jnp.int32),
        pltpu.VMEM((GSUB, 128), jnp.int32),
        pltpu.VMEM((CHUNK, WD), jnp.float32),
        pltpu.VMEM((CHUNK, PD), jnp.float32),
        pltpu.VMEM((CHUNK, PD), jnp.float32),
        pltpu.SemaphoreType.DMA,
    ],
    compiler_params=pltpu.CompilerParams(use_tc_tiling_on_sc=False,
                                         needs_layout_passes=False),
)
def _embed_sc(words_hbm, head_hbm, tail_hbm, wt_hbm, ht_hbm, tt_hbm,
              out_hbm, widx_v, hidx_v, tidx_v, wrow_v, hrow_v, trow_v, sem):
    wid = lax.axis_index("s") * NC + lax.axis_index("c")
    row0 = wid * (N_PER_W // 128)

    def chunk_body(ci, _):
        r = row0 + ci * GSUB
        tok0 = r * 128

        pltpu.sync_copy(words_hbm.at[pl.ds(r, GSUB)], widx_v)
        pltpu.sync_copy(head_hbm.at[pl.ds(r, GSUB)], hidx_v)
        pltpu.sync_copy(tail_hbm.at[pl.ds(r, GSUB)], tidx_v)

        copies = []
        for j in range(GSUB):
            dst = pl.ds(j * 128, 128)
            copies.append(
                pltpu.async_copy(wt_hbm.at[widx_v.at[j]], wrow_v.at[dst], sem))
            copies.append(
                pltpu.async_copy(ht_hbm.at[hidx_v.at[j]], hrow_v.at[dst], sem))
            copies.append(
                pltpu.async_copy(tt_hbm.at[tidx_v.at[j]], trow_v.at[dst], sem))
        for c in copies:
            c.wait()

        def fixup_body(j, _):
            for o in range(128 // L):
                idxs = widx_v[j, pl.ds(o * L, L)]
                msk = idxs == 0

                @pl.when(jnp.min(idxs) == 0)
                def _():
                    toks = j * 128 + o * L + lax.iota(jnp.int32, L)
                    zf = jnp.zeros((L,), jnp.float32)
                    for col in range(WD):
                        plsc.store_scatter(
                            wrow_v, [toks, jnp.full((L,), col, jnp.int32)],
                            zf, mask=msk)
            return 0

        lax.fori_loop(0, GSUB, fixup_body, 0)

        pltpu.sync_copy(wrow_v, out_hbm.at[pl.ds(tok0, CHUNK), pl.ds(0, WD)])
        pltpu.sync_copy(hrow_v, out_hbm.at[pl.ds(tok0, CHUNK), pl.ds(WD, PD)])
        pltpu.sync_copy(trow_v, out_hbm.at[pl.ds(tok0, CHUNK), pl.ds(WD + PD, PD)])
        return 0

    lax.fori_loop(0, NCHUNK, chunk_body, 0)


def kernel(words, head_pos, tail_pos, word_table, head_pos_table, tail_pos_table):
    ht = head_pos_table.at[0].set(0.0)
    tt = tail_pos_table.at[0].set(0.0)
    w2 = words.reshape(NROW128, 128)
    h2 = head_pos.reshape(NROW128, 128)
    t2 = tail_pos.reshape(NROW128, 128)
    out = _embed_sc(w2, h2, t2, word_table, ht, tt)
    return out.reshape(B, SEQ, OD)
